# Initial kernel scaffold; baseline (speedup 1.0000x reference)
import jax
import jax.numpy as jnp
from jax.experimental import pallas as pl


def kernel(initial_position, contact_node, parent2child, branch, contact_force, edge_index, batch, params):
    raise NotImplementedError("write your pallas kernel here")


# speedup vs baseline: 2.0625x; 2.0625x over previous
import functools

import jax
import jax.numpy as jnp
from jax import lax
from jax.experimental import pallas as pl
from jax.experimental.pallas import tpu as pltpu
from jax.experimental.pallas import tpu_sc as plsc

H = 128
_NC = 2
_NS = 16
_NW = _NC * _NS
_K = 128
_BE = 2000
_BN = 2000

_F32 = jnp.float32


def _ln(x, g, b):
    mu = jnp.mean(x, axis=-1, keepdims=True)
    xc = x - mu
    var = jnp.mean(xc * xc, axis=-1, keepdims=True)
    return xc * lax.rsqrt(var + 1e-5) * g + b


def _dot(a, b):
    return jnp.dot(a, b, preferred_element_type=_F32,
                   precision=lax.Precision.HIGHEST)


def _full(shape):
    return pl.BlockSpec(shape, lambda i: tuple(0 for _ in shape))


@functools.lru_cache(maxsize=None)
def _sc_gather_fn(N, E):
    nchunks = E // _K
    assert nchunks * _K == E
    mesh = plsc.VectorSubcoreMesh(core_axis_name="c", subcore_axis_name="s")

    @functools.partial(
        pl.kernel,
        mesh=mesh,
        out_type=jax.ShapeDtypeStruct((E, H), _F32),
        scratch_types=[
            pltpu.VMEM((_K,), jnp.int32),
            pltpu.VMEM((_K,), jnp.int32),
            pltpu.VMEM((_K, H), _F32),
            pltpu.SemaphoreType.DMA,
        ],
    )
    def gather(p_hbm, q_hbm, dst_hbm, src_hbm, out_hbm, idx_d, idx_s, buf, sem):
        wid = lax.axis_index("c") * _NS + lax.axis_index("s")
        trips = (nchunks - wid + _NW - 1) // _NW

        def it(i, carry):
            base = (i * _NW + wid) * _K
            pltpu.sync_copy(dst_hbm.at[pl.ds(base, _K)], idx_d)
            pltpu.sync_copy(src_hbm.at[pl.ds(base, _K)], idx_s)
            pltpu.async_copy(p_hbm.at[idx_d], buf, sem).wait()
            pltpu.async_copy(q_hbm.at[idx_s], buf, sem, add=True).wait()
            pltpu.sync_copy(buf, out_hbm.at[pl.ds(base, _K)])
            return carry

        lax.fori_loop(0, trips, it, 0, unroll=False)

    return gather


@functools.lru_cache(maxsize=None)
def _sc_scatter_fn(N, E):
    nchunks = E // _K
    assert nchunks * _K == E
    rows = (N // (8 * _NS)) * 8
    tail0 = rows * _NS
    tail = N - tail0
    mesh = plsc.VectorSubcoreMesh(core_axis_name="c", subcore_axis_name="s")

    @functools.partial(
        pl.kernel,
        mesh=mesh,
        out_type=jax.ShapeDtypeStruct((_NC, N, H), _F32),
        scratch_types=[
            pltpu.VMEM((_K,), jnp.int32),
            pltpu.VMEM((_K, H), _F32),
            pltpu.VMEM_SHARED((N, H), _F32),
            pltpu.SemaphoreType.DMA,
        ],
    )
    def scatter(m_hbm, dst_hbm, zeros_hbm, out_hbm, idx_v, mbuf, acc_sh, sem):
        cid = lax.axis_index("c")
        sid = lax.axis_index("s")
        wid = cid * _NS + sid
        r0 = sid * rows
        pltpu.sync_copy(zeros_hbm.at[pl.ds(r0, rows)], acc_sh.at[pl.ds(r0, rows)])
        if tail:
            @pl.when(sid == _NS - 1)
            def _():
                pltpu.sync_copy(zeros_hbm.at[pl.ds(tail0, tail)],
                                acc_sh.at[pl.ds(tail0, tail)])
        plsc.subcore_barrier()
        trips = (nchunks - wid + _NW - 1) // _NW

        def it(i, carry):
            base = (i * _NW + wid) * _K
            pltpu.sync_copy(dst_hbm.at[pl.ds(base, _K)], idx_v)
            pltpu.sync_copy(m_hbm.at[pl.ds(base, _K)], mbuf)
            pltpu.sync_copy(mbuf, acc_sh.at[idx_v], add=True)
            return carry

        lax.fori_loop(0, trips, it, 0, unroll=False)
        plsc.subcore_barrier()
        pltpu.sync_copy(acc_sh.at[pl.ds(r0, rows)], out_hbm.at[cid, pl.ds(r0, rows)])
        if tail:
            @pl.when(sid == _NS - 1)
            def _():
                pltpu.sync_copy(acc_sh.at[pl.ds(tail0, tail)],
                                out_hbm.at[cid, pl.ds(tail0, tail)])

    return scatter


def _encoder_call(x8, batch2, gf8, wn, bxs):
    N = x8.shape[0]
    G = gf8.shape[0]
    (w1, b1), (w2, b2), (w3, b3) = wn
    wxi, wxj, wgf = bxs
    nblk = N // _BN

    def body(x_r, bt_r, gf_r, w1r, b1r, w2r, b2r, w3r, b3r, wxir, wxjr, wgfr,
             nf_o, gfn_o, p_o, q_o):
        h = jnp.maximum(_dot(x_r[...], w1r[...]) + b1r[...], 0.0)
        h = jnp.maximum(_dot(h, w2r[...]) + b2r[...], 0.0)
        nf = _dot(h, w3r[...]) + b3r[...]
        onehot = (bt_r[...] == lax.broadcasted_iota(jnp.int32, (1, G), 1)).astype(_F32)
        gfn = _dot(onehot, gf_r[...])
        nf_o[...] = nf
        gfn_o[...] = gfn
        p_o[...] = _dot(nf, wxir[...])
        q_o[...] = _dot(nf, wxjr[...]) + _dot(gfn, wgfr[...])

    return pl.pallas_call(
        body,
        grid=(nblk,),
        in_specs=[
            pl.BlockSpec((_BN, 8), lambda i: (i, 0)),
            pl.BlockSpec((_BN, 1), lambda i: (i, 0)),
            _full((G, 8)),
            _full((8, H)), _full((1, H)),
            _full((H, H)), _full((1, H)),
            _full((H, H)), _full((1, H)),
            _full((H, H)), _full((H, H)), _full((8, H)),
        ],
        out_specs=[
            pl.BlockSpec((_BN, H), lambda i: (i, 0)),
            pl.BlockSpec((_BN, 8), lambda i: (i, 0)),
            pl.BlockSpec((_BN, H), lambda i: (i, 0)),
            pl.BlockSpec((_BN, H), lambda i: (i, 0)),
        ],
        out_shape=[
            jax.ShapeDtypeStruct((N, H), _F32),
            jax.ShapeDtypeStruct((N, 8), _F32),
            jax.ShapeDtypeStruct((N, H), _F32),
            jax.ShapeDtypeStruct((N, H), _F32),
        ],
    )(x8, batch2, gf8, w1, b1, w2, b2, w3, b3, wxi, wxj, wgf)


def _edge_call(gath, ef_or_ea8, wm, first, want_ef, we_in=None):
    E = gath.shape[0]
    nblk = E // _BE
    wc, b1, w2, b2, w3, b3, g, bt = wm

    def mlp_tail(efv, gv, wcr, b1r, w2r, b2r, w3r, b3r, gr, btr):
        z = jnp.maximum(gv + _dot(efv, wcr[...]) + b1r[...], 0.0)
        z = jnp.maximum(_dot(z, w2r[...]) + b2r[...], 0.0)
        m = _dot(z, w3r[...]) + b3r[...]
        return _ln(m, gr[...], btr[...])

    wspecs = [_full((H, H)), _full((1, H)), _full((H, H)), _full((1, H)),
              _full((H, H)), _full((1, H)), _full((1, H)), _full((1, H))]
    out_specs = [pl.BlockSpec((_BE, H), lambda i: (i, 0))]
    out_shape = [jax.ShapeDtypeStruct((E, H), _F32)]
    if want_ef:
        out_specs.append(pl.BlockSpec((_BE, H), lambda i: (i, 0)))
        out_shape.append(jax.ShapeDtypeStruct((E, H), _F32))

    if first:
        (e1, eb1), (e2, eb2), (e3, eb3) = we_in

        def body(g_r, ea_r, e1r, eb1r, e2r, eb2r, e3r, eb3r,
                 wcr, b1r, w2r, b2r, w3r, b3r, gr, btr, m_o, ef_o):
            h = jnp.maximum(_dot(ea_r[...], e1r[...]) + eb1r[...], 0.0)
            h = jnp.maximum(_dot(h, e2r[...]) + eb2r[...], 0.0)
            ef = _dot(h, e3r[...]) + eb3r[...]
            m = mlp_tail(ef, g_r[...], wcr, b1r, w2r, b2r, w3r, b3r, gr, btr)
            m_o[...] = m
            ef_o[...] = ef + m

        return pl.pallas_call(
            body,
            grid=(nblk,),
            in_specs=[
                pl.BlockSpec((_BE, H), lambda i: (i, 0)),
                pl.BlockSpec((_BE, 8), lambda i: (i, 0)),
                _full((8, H)), _full((1, H)),
                _full((H, H)), _full((1, H)),
                _full((H, H)), _full((1, H)),
            ] + wspecs,
            out_specs=out_specs,
            out_shape=out_shape,
        )(gath, ef_or_ea8, e1, eb1, e2, eb2, e3, eb3, wc, b1, w2, b2, w3, b3, g, bt)

    if want_ef:
        def body(g_r, ef_r, wcr, b1r, w2r, b2r, w3r, b3r, gr, btr, m_o, ef_o):
            ef = ef_r[...]
            m = mlp_tail(ef, g_r[...], wcr, b1r, w2r, b2r, w3r, b3r, gr, btr)
            m_o[...] = m
            ef_o[...] = ef + m
    else:
        def body(g_r, ef_r, wcr, b1r, w2r, b2r, w3r, b3r, gr, btr, m_o):
            m = mlp_tail(ef_r[...], g_r[...], wcr, b1r, w2r, b2r, w3r, b3r, gr, btr)
            m_o[...] = m

    return pl.pallas_call(
        body,
        grid=(nblk,),
        in_specs=[
            pl.BlockSpec((_BE, H), lambda i: (i, 0)),
            pl.BlockSpec((_BE, H), lambda i: (i, 0)),
        ] + wspecs,
        out_specs=out_specs,
        out_shape=out_shape,
    )(gath, ef_or_ea8, wc, b1, w2, b2, w3, b3, g, bt)


def _node_call(nf, part, gfn8, wm, nxt=None, wout=None):
    N = nf.shape[0]
    nblk = N // _BN
    w1a, w1b, w1g, b1, w2, b2, w3, b3, g, bt = wm

    def trunk(nf_v, part_v, gfn_v, w1ar, w1br, w1gr, b1r, w2r, b2r, w3r, b3r, gr, btr):
        aggr = part_v[0] + part_v[1]
        h = _dot(nf_v, w1ar[...]) + _dot(aggr, w1br[...]) + _dot(gfn_v, w1gr[...]) + b1r[...]
        h = jnp.maximum(h, 0.0)
        h = jnp.maximum(_dot(h, w2r[...]) + b2r[...], 0.0)
        o = _dot(h, w3r[...]) + b3r[...]
        return nf_v + _ln(o, gr[...], btr[...])

    base_specs = [
        pl.BlockSpec((_BN, H), lambda i: (i, 0)),
        pl.BlockSpec((_NC, _BN, H), lambda i: (0, i, 0)),
        pl.BlockSpec((_BN, 8), lambda i: (i, 0)),
        _full((H, H)), _full((H, H)), _full((8, H)), _full((1, H)),
        _full((H, H)), _full((1, H)), _full((H, H)), _full((1, H)),
        _full((1, H)), _full((1, H)),
    ]

    if nxt is not None:
        wxi, wxj, wgf = nxt

        def body(nf_r, part_r, gfn_r, w1ar, w1br, w1gr, b1r, w2r, b2r, w3r, b3r,
                 gr, btr, wxir, wxjr, wgfr, nf_o, p_o, q_o):
            nf2 = trunk(nf_r[...], part_r[...], gfn_r[...], w1ar, w1br, w1gr,
                        b1r, w2r, b2r, w3r, b3r, gr, btr)
            nf_o[...] = nf2
            p_o[...] = _dot(nf2, wxir[...])
            q_o[...] = _dot(nf2, wxjr[...]) + _dot(gfn_r[...], wgfr[...])

        return pl.pallas_call(
            body,
            grid=(nblk,),
            in_specs=base_specs + [_full((H, H)), _full((H, H)), _full((8, H))],
            out_specs=[
                pl.BlockSpec((_BN, H), lambda i: (i, 0)),
                pl.BlockSpec((_BN, H), lambda i: (i, 0)),
                pl.BlockSpec((_BN, H), lambda i: (i, 0)),
            ],
            out_shape=[
                jax.ShapeDtypeStruct((N, H), _F32),
                jax.ShapeDtypeStruct((N, H), _F32),
                jax.ShapeDtypeStruct((N, H), _F32),
            ],
        )(nf, part, gfn8, w1a, w1b, w1g, b1, w2, b2, w3, b3, g, bt, wxi, wxj, wgf)

    (o1, ob1), (o2, ob2), (o3, ob3) = wout

    def body(nf_r, part_r, gfn_r, w1ar, w1br, w1gr, b1r, w2r, b2r, w3r, b3r,
             gr, btr, o1r, ob1r, o2r, ob2r, o3r, ob3r, out_o):
        nf2 = trunk(nf_r[...], part_r[...], gfn_r[...], w1ar, w1br, w1gr,
                    b1r, w2r, b2r, w3r, b3r, gr, btr)
        h = jnp.maximum(_dot(nf2, o1r[...]) + ob1r[...], 0.0)
        h = jnp.maximum(_dot(h, o2r[...]) + ob2r[...], 0.0)
        out_o[...] = _dot(h, o3r[...]) + ob3r[...]

    dim = o3.shape[1]
    return pl.pallas_call(
        body,
        grid=(nblk,),
        in_specs=base_specs + [_full((H, H)), _full((1, H)),
                               _full((H, H)), _full((1, H)),
                               _full((H, dim)), _full((1, dim))],
        out_specs=pl.BlockSpec((_BN, dim), lambda i: (i, 0)),
        out_shape=jax.ShapeDtypeStruct((N, dim), _F32),
    )(nf, part, gfn8, w1a, w1b, w1g, b1, w2, b2, w3, b3, g, bt,
      o1, ob1, o2, ob2, o3, ob3)


def _pad_lanes(a, lanes):
    return jnp.pad(a, ((0, 0), (0, lanes - a.shape[1])))


def _row(b):
    return b.reshape(1, -1)


def kernel(initial_position, contact_node, parent2child, branch, contact_force,
           edge_index, batch, params):
    N = initial_position.shape[0]
    E = edge_index.shape[1]
    src = edge_index[0]
    dst = edge_index[1]
    gf = contact_force.reshape(-1, 3)
    gf8 = _pad_lanes(gf, 8)
    x8 = _pad_lanes(
        jnp.concatenate([initial_position, contact_node[:, None]], axis=-1), 8)
    ea8 = _pad_lanes(
        jnp.concatenate([parent2child[:, None], branch[:, None]], axis=-1), 8)
    batch2 = batch[:, None]
    zeros_nh = jnp.zeros((N, H), _F32)

    def lin8(Wb):
        W, b = Wb
        return jnp.pad(W, ((0, 8 - W.shape[0]), (0, 0))), _row(b)

    wn_in = [lin8(params["node_in"][0]),
             (params["node_in"][1][0], _row(params["node_in"][1][1])),
             (params["node_in"][2][0], _row(params["node_in"][2][1]))]
    we_in = [lin8(params["edge_in"][0]),
             (params["edge_in"][1][0], _row(params["edge_in"][1][1])),
             (params["edge_in"][2][0], _row(params["edge_in"][2][1]))]
    wout = [(W, _row(b)) for (W, b) in params["node_out"]]

    esplit, nsplit = [], []
    for layer in params["IN"]:
        (w1, b1), (w2, b2), (w3, b3) = layer["edge"]["lin"]
        g, bt = layer["edge"]["ln"]
        wxi = w1[:H]
        wxj = w1[H:2 * H]
        wc = w1[2 * H:3 * H]
        wgf = jnp.pad(w1[3 * H:], ((0, 5), (0, 0)))
        esplit.append({
            "pq": (wxi, wxj, wgf),
            "m": (wc, _row(b1), w2, _row(b2), w3, _row(b3), _row(g), _row(bt)),
        })
        (nw1, nb1), (nw2, nb2), (nw3, nb3) = layer["node"]["lin"]
        ng, nbt = layer["node"]["ln"]
        nsplit.append((nw1[:H], nw1[H:2 * H],
                       jnp.pad(nw1[2 * H:], ((0, 5), (0, 0))),
                       _row(nb1), nw2, _row(nb2), nw3, _row(nb3),
                       _row(ng), _row(nbt)))

    gather = _sc_gather_fn(N, E)
    scatter = _sc_scatter_fn(N, E)

    nf, gfn8, P, Q = _encoder_call(x8, batch2, gf8, wn_in, esplit[0]["pq"])

    ef = None
    out = None
    L = len(params["IN"])
    for l in range(L):
        G = gather(P, Q, dst, src)
        if l == 0:
            m, ef = _edge_call(G, ea8, esplit[l]["m"], first=True, want_ef=True,
                               we_in=we_in)
        elif l < L - 1:
            m, ef = _edge_call(G, ef, esplit[l]["m"], first=False, want_ef=True)
        else:
            (m,) = _edge_call(G, ef, esplit[l]["m"], first=False, want_ef=False)
        part = scatter(m, dst, zeros_nh)
        if l < L - 1:
            nf, P, Q = _node_call(nf, part, gfn8, nsplit[l],
                                  nxt=esplit[l + 1]["pq"])
        else:
            out = _node_call(nf, part, gfn8, nsplit[l], wout=wout)
    return out
